# Initial kernel scaffold; baseline (speedup 1.0000x reference)
import jax
import jax.numpy as jnp
from jax.experimental import pallas as pl


def kernel(x, pos_emb):
    raise NotImplementedError("write your pallas kernel here")


# speedup vs baseline: 1.6947x; 1.6947x over previous
import jax
import jax.numpy as jnp
from jax.experimental import pallas as pl
from jax.experimental.pallas import tpu as pltpu

_BLOCK_S = 512


def _pe_add_kernel(x_ref, pe_ref, o_ref):
    o_ref[...] = x_ref[...] + pe_ref[...][None, :, :]


def kernel(x, pos_emb):
    b, s, d = x.shape
    bs = _BLOCK_S if s % _BLOCK_S == 0 else s
    grid = (s // bs, b)
    return pl.pallas_call(
        _pe_add_kernel,
        grid=grid,
        in_specs=[
            pl.BlockSpec((1, bs, d), lambda i, j: (j, i, 0)),
            pl.BlockSpec((bs, d), lambda i, j: (i, 0)),
        ],
        out_specs=pl.BlockSpec((1, bs, d), lambda i, j: (j, i, 0)),
        out_shape=jax.ShapeDtypeStruct((b, s, d), x.dtype),
        compiler_params=pltpu.CompilerParams(
            dimension_semantics=("arbitrary", "arbitrary"),
        ),
    )(x, pos_emb)


# speedup vs baseline: 1.8793x; 1.1089x over previous
import jax
import jax.numpy as jnp
from jax.experimental import pallas as pl
from jax.experimental.pallas import tpu as pltpu

_BLOCK_S = 1024


def _pe_add_kernel(x_ref, pe_ref, o_ref):
    o_ref[...] = x_ref[...] + pe_ref[...][None, :, :]


def kernel(x, pos_emb):
    b, s, d = x.shape
    bs = _BLOCK_S if s % _BLOCK_S == 0 else s
    grid = (s // bs, b)
    return pl.pallas_call(
        _pe_add_kernel,
        grid=grid,
        in_specs=[
            pl.BlockSpec((1, bs, d), lambda i, j: (j, i, 0)),
            pl.BlockSpec((bs, d), lambda i, j: (i, 0)),
        ],
        out_specs=pl.BlockSpec((1, bs, d), lambda i, j: (j, i, 0)),
        out_shape=jax.ShapeDtypeStruct((b, s, d), x.dtype),
        compiler_params=pltpu.CompilerParams(
            dimension_semantics=("arbitrary", "arbitrary"),
        ),
    )(x, pos_emb)


# speedup vs baseline: 1.9912x; 1.0595x over previous
import jax
import jax.numpy as jnp
from jax.experimental import pallas as pl
from jax.experimental.pallas import tpu as pltpu

_BLOCK_S = 2048


def _pe_add_kernel(x_ref, pe_ref, o_ref):
    o_ref[...] = x_ref[...] + pe_ref[...][None, :, :]


def kernel(x, pos_emb):
    b, s, d = x.shape
    bs = _BLOCK_S if s % _BLOCK_S == 0 else s
    grid = (s // bs, b)
    return pl.pallas_call(
        _pe_add_kernel,
        grid=grid,
        in_specs=[
            pl.BlockSpec((1, bs, d), lambda i, j: (j, i, 0)),
            pl.BlockSpec((bs, d), lambda i, j: (i, 0)),
        ],
        out_specs=pl.BlockSpec((1, bs, d), lambda i, j: (j, i, 0)),
        out_shape=jax.ShapeDtypeStruct((b, s, d), x.dtype),
        compiler_params=pltpu.CompilerParams(
            dimension_semantics=("arbitrary", "arbitrary"),
        ),
    )(x, pos_emb)


# speedup vs baseline: 1.9969x; 1.0028x over previous
import jax
import jax.numpy as jnp
from jax.experimental import pallas as pl
from jax.experimental.pallas import tpu as pltpu

_BLOCK_S = 2048


def _pe_add_kernel(x_ref, pe_ref, o_ref):
    o_ref[...] = x_ref[...] + pe_ref[...][None, :, :]


def kernel(x, pos_emb):
    b, s, d = x.shape
    bs = _BLOCK_S if s % _BLOCK_S == 0 else s
    grid = (s // bs, b)
    return pl.pallas_call(
        _pe_add_kernel,
        grid=grid,
        in_specs=[
            pl.BlockSpec((1, bs, d), lambda i, j: (j, i, 0)),
            pl.BlockSpec((bs, d), lambda i, j: (i, 0)),
        ],
        out_specs=pl.BlockSpec((1, bs, d), lambda i, j: (j, i, 0)),
        out_shape=jax.ShapeDtypeStruct((b, s, d), x.dtype),
        compiler_params=pltpu.CompilerParams(
            dimension_semantics=("parallel", "parallel"),
        ),
    )(x, pos_emb)
